# Initial kernel scaffold; baseline (speedup 1.0000x reference)
import jax
import jax.numpy as jnp
from jax.experimental import pallas as pl


def kernel(dynamic, static, mask, W_c0, b_c0, W_c1, b_c1, W_c2, b_c2, W_r0, W_r1, grn_W1, grn_b1, grn_W2, grn_b2, grn_Wg, grn_bg, grn_Ws, grn_bs, ln_g, ln_b, db, W_fuse, b_fuse, W_h1, b_h1, W_h2, b_h2):
    raise NotImplementedError("write your pallas kernel here")


# speedup vs baseline: 1.4820x; 1.4820x over previous
import jax
import jax.numpy as jnp
from jax.experimental import pallas as pl

_BB_TCN = 64
_BB_HEAD = 128
_TOPK = 5


def _relu(x):
    return jnp.maximum(x, 0.0)


def _dot(a, b):
    return jnp.dot(a, b, preferred_element_type=jnp.float32)


def _tcn_kernel(dyn_ref, mask_ref, wc0_ref, wc1_ref, wc2_ref, wr0_ref,
                wr1_ref, b0_ref, b1_ref, b2_ref, hcur_ref, hglob_ref):
    BB, T, ND = dyn_ref.shape
    H = hcur_ref.shape[-1]
    x = dyn_ref[...].reshape(BB * T, ND)
    tmod = jax.lax.broadcasted_iota(jnp.int32, (BB * T, 1), 0) % T

    def shifted(xf, s):
        z = jnp.zeros((s, xf.shape[1]), xf.dtype)
        xs = jnp.concatenate([z, xf[:-s]], axis=0)
        return jnp.where(tmod >= s, xs, 0.0)

    def causal(xf, wc, bias, d):
        y = _dot(xf, wc[2])
        y = y + _dot(shifted(xf, d), wc[1])
        y = y + _dot(shifted(xf, 2 * d), wc[0])
        return y + bias

    h = _relu(causal(x, wc0_ref[...], b0_ref[...], 1))
    x1 = _relu(h + _dot(x, wr0_ref[...]))
    h = _relu(causal(x1, wc1_ref[...], b1_ref[...], 2))
    x2 = _relu(h + _dot(x1, wr1_ref[...]))
    h = _relu(causal(x2, wc2_ref[...], b2_ref[...], 4))
    x3 = _relu(h + x2)

    m = mask_ref[...]
    x3d = x3.reshape(BB, T, H)
    msum = jnp.sum(m, axis=1, keepdims=True)
    hglob_ref[...] = (jnp.sum(x3d * m[:, :, None], axis=1)
                      / jnp.maximum(msum, 1.0))
    lengths = jnp.maximum(msum.astype(jnp.int32) - 1, 0)
    tio = jax.lax.broadcasted_iota(jnp.int32, (BB, T), 1)
    sel = (tio == lengths).astype(jnp.float32)
    hcur_ref[...] = jnp.sum(x3d * sel[:, :, None], axis=1)


def _head_kernel(hcur_ref, hglob_ref, static_ref, db_ref, w1_ref, bb1_ref,
                 w2_ref, bb2_ref, wg_ref, bg_ref, ws_ref, bs_ref, lng_ref,
                 lnb_ref, wf_ref, bf_ref, wh1_ref, bh1_ref, wh2_ref, bh2_ref,
                 pred_ref):
    BB = hcur_ref.shape[0]
    H = hcur_ref.shape[1]
    db = db_ref[...]
    NP = db.shape[0]
    hc = hcur_ref[...]

    sim = jax.lax.dot_general(hc, db, (((1,), (1,)), ((), ())),
                              preferred_element_type=jnp.float32)
    tio = jax.lax.broadcasted_iota(jnp.int32, (BB, NP), 1)
    s = sim
    v0 = None
    wmat = jnp.zeros((BB, NP), jnp.float32)
    z = None
    for k in range(_TOPK):
        mx = jnp.max(s, axis=1, keepdims=True)
        if k == 0:
            v0 = mx
            we = jnp.ones_like(mx)
        else:
            we = jnp.exp(mx - v0)
        eq = s == mx
        idx = jnp.min(jnp.where(eq, tio, NP), axis=1, keepdims=True)
        hot = tio == idx
        wmat = jnp.where(hot, we, wmat)
        s = jnp.where(hot, -1e30, s)
        z = we if k == 0 else z + we
    agg = _dot(wmat, db) / z

    st = static_ref[...]
    h1p = _dot(st, w1_ref[...]) + bb1_ref[...]
    h1 = jnp.where(h1p > 0, h1p, jnp.exp(jnp.minimum(h1p, 0.0)) - 1.0)
    h2 = _dot(h1, w2_ref[...]) + bb2_ref[...]
    g = _dot(h2, wg_ref[...]) + bg_ref[...]
    a = g[:, :H]
    bv = g[:, H:]
    gated = bv / (1.0 + jnp.exp(-a))
    skip = _dot(st, ws_ref[...]) + bs_ref[...]
    y = skip + gated
    mu = jnp.mean(y, axis=-1, keepdims=True)
    var = jnp.mean((y - mu) ** 2, axis=-1, keepdims=True)
    h_static = (y - mu) / jnp.sqrt(var + 1e-5) * lng_ref[...] + lnb_ref[...]

    wf = wf_ref[...]
    h_f = _dot(hc, wf[:H]) + _dot(agg, wf[H:]) + bf_ref[...]
    zin = h_f + h_static
    wh1 = wh1_ref[...]
    hh = _relu(_dot(zin, wh1[:H]) + _dot(hglob_ref[...], wh1[H:])
               + bh1_ref[...])
    pred_ref[...] = _dot(hh, wh2_ref[...]) + bh2_ref[...]


def kernel(dynamic, static, mask, W_c0, b_c0, W_c1, b_c1, W_c2, b_c2, W_r0,
           W_r1, grn_W1, grn_b1, grn_W2, grn_b2, grn_Wg, grn_bg, grn_Ws,
           grn_bs, ln_g, ln_b, db, W_fuse, b_fuse, W_h1, b_h1, W_h2, b_h2):
    B, T, ND = dynamic.shape
    H = db.shape[1]
    NP = db.shape[0]

    wc0 = jnp.transpose(W_c0, (2, 1, 0))
    wc1 = jnp.transpose(W_c1, (2, 1, 0))
    wc2 = jnp.transpose(W_c2, (2, 1, 0))
    wr0 = W_r0[:, :, 0].T
    wr1 = W_r1[:, :, 0].T
    row = lambda v: v.reshape(1, -1)

    bb = _BB_TCN
    grid_a = (B // bb,)
    const3 = lambda shp: pl.BlockSpec(shp, lambda i: (0, 0, 0))
    const2 = lambda shp: pl.BlockSpec(shp, lambda i: (0, 0))
    hcur, hglob = pl.pallas_call(
        _tcn_kernel,
        grid=grid_a,
        in_specs=[
            pl.BlockSpec((bb, T, ND), lambda i: (i, 0, 0)),
            pl.BlockSpec((bb, T), lambda i: (i, 0)),
            const3(wc0.shape), const3(wc1.shape), const3(wc2.shape),
            const2(wr0.shape), const2(wr1.shape),
            const2((1, 64)), const2((1, 128)), const2((1, 128)),
        ],
        out_specs=[
            pl.BlockSpec((bb, H), lambda i: (i, 0)),
            pl.BlockSpec((bb, H), lambda i: (i, 0)),
        ],
        out_shape=[
            jax.ShapeDtypeStruct((B, H), jnp.float32),
            jax.ShapeDtypeStruct((B, H), jnp.float32),
        ],
    )(dynamic, mask, wc0, wc1, wc2, wr0, wr1,
      row(b_c0), row(b_c1), row(b_c2))

    bh = _BB_HEAD
    grid_b = (B // bh,)
    pred = pl.pallas_call(
        _head_kernel,
        grid=grid_b,
        in_specs=[
            pl.BlockSpec((bh, H), lambda i: (i, 0)),
            pl.BlockSpec((bh, H), lambda i: (i, 0)),
            pl.BlockSpec((bh, static.shape[1]), lambda i: (i, 0)),
            const2(db.shape),
            const2(grn_W1.shape), const2((1, 64)),
            const2(grn_W2.shape), const2((1, 128)),
            const2(grn_Wg.shape), const2((1, 256)),
            const2(grn_Ws.shape), const2((1, 128)),
            const2((1, 128)), const2((1, 128)),
            const2(W_fuse.shape), const2((1, 128)),
            const2(W_h1.shape), const2((1, 128)),
            const2(W_h2.shape), const2((1, 1)),
        ],
        out_specs=pl.BlockSpec((bh, 1), lambda i: (i, 0)),
        out_shape=jax.ShapeDtypeStruct((B, 1), jnp.float32),
    )(hcur, hglob, static, db, grn_W1, row(grn_b1), grn_W2, row(grn_b2),
      grn_Wg, row(grn_bg), grn_Ws, row(grn_bs), row(ln_g), row(ln_b),
      W_fuse, row(b_fuse), W_h1, row(b_h1), W_h2, row(b_h2))

    return pred[:, 0], hcur


# speedup vs baseline: 1.5937x; 1.0753x over previous
import jax
import jax.numpy as jnp
from jax.experimental import pallas as pl

_BB_TCN = 64
_BB_HEAD = 128
_TOPK = 5


def _relu(x):
    return jnp.maximum(x, 0.0)


def _dot(a, b):
    return jnp.dot(a, b, preferred_element_type=jnp.float32)


def _tcn_kernel(dyn_ref, mask_ref, wc0_ref, wc1_ref, wc2_ref, wr0_ref,
                wr1_ref, b0_ref, b1_ref, b2_ref, hcur_ref, hglob_ref):
    BB, T, ND = dyn_ref.shape
    H = hcur_ref.shape[-1]
    x = dyn_ref[...].reshape(BB * T, ND)
    tmod = jax.lax.broadcasted_iota(jnp.int32, (BB * T, 1), 0) % T

    def shifted(xf, s):
        z = jnp.zeros((s, xf.shape[1]), xf.dtype)
        xs = jnp.concatenate([z, xf[:-s]], axis=0)
        return jnp.where(tmod >= s, xs, 0.0)

    def causal(xf, wc, bias, d):
        y = _dot(xf, wc[2])
        y = y + _dot(shifted(xf, d), wc[1])
        y = y + _dot(shifted(xf, 2 * d), wc[0])
        return y + bias

    h = _relu(causal(x, wc0_ref[...], b0_ref[...], 1))
    x1 = _relu(h + _dot(x, wr0_ref[...]))
    h = _relu(causal(x1, wc1_ref[...], b1_ref[...], 2))
    x2 = _relu(h + _dot(x1, wr1_ref[...]))
    h = _relu(causal(x2, wc2_ref[...], b2_ref[...], 4))
    x3 = _relu(h + x2)

    m = mask_ref[...]
    x3d = x3.reshape(BB, T, H)
    msum = jnp.sum(m, axis=1, keepdims=True)
    hglob_ref[...] = (jnp.sum(x3d * m[:, :, None], axis=1)
                      / jnp.maximum(msum, 1.0))
    lengths = jnp.maximum(msum.astype(jnp.int32) - 1, 0)
    tio = jax.lax.broadcasted_iota(jnp.int32, (BB, T), 1)
    sel = (tio == lengths).astype(jnp.float32)
    hcur_ref[...] = jnp.sum(x3d * sel[:, :, None], axis=1)


def _head_kernel(hcur_ref, hglob_ref, static_ref, db_ref, w1_ref, bb1_ref,
                 w2_ref, bb2_ref, wg_ref, bg_ref, ws_ref, bs_ref, lng_ref,
                 lnb_ref, wf_ref, bf_ref, wh1_ref, bh1_ref, wh2_ref, bh2_ref,
                 pred_ref):
    BB = hcur_ref.shape[0]
    H = hcur_ref.shape[1]
    db = db_ref[...]
    NP = db.shape[0]
    hc = hcur_ref[...]

    sim = jax.lax.dot_general(hc, db, (((1,), (1,)), ((), ())),
                              preferred_element_type=jnp.float32)
    s = sim
    v0 = None
    wmat = jnp.zeros((BB, NP), jnp.float32)
    z = None
    for k in range(_TOPK):
        mx = jnp.max(s, axis=1, keepdims=True)
        if k == 0:
            v0 = mx
            we = jnp.ones_like(mx)
        else:
            we = jnp.exp(mx - v0)
        hot = s == mx
        wmat = jnp.where(hot, we, wmat)
        s = jnp.where(hot, -1e30, s)
        z = we if k == 0 else z + we
    agg = _dot(wmat, db) / z

    st = static_ref[...]
    h1p = _dot(st, w1_ref[...]) + bb1_ref[...]
    h1 = jnp.where(h1p > 0, h1p, jnp.exp(jnp.minimum(h1p, 0.0)) - 1.0)
    h2 = _dot(h1, w2_ref[...]) + bb2_ref[...]
    g = _dot(h2, wg_ref[...]) + bg_ref[...]
    a = g[:, :H]
    bv = g[:, H:]
    gated = bv / (1.0 + jnp.exp(-a))
    skip = _dot(st, ws_ref[...]) + bs_ref[...]
    y = skip + gated
    mu = jnp.mean(y, axis=-1, keepdims=True)
    var = jnp.mean((y - mu) ** 2, axis=-1, keepdims=True)
    h_static = (y - mu) / jnp.sqrt(var + 1e-5) * lng_ref[...] + lnb_ref[...]

    wf = wf_ref[...]
    h_f = _dot(hc, wf[:H]) + _dot(agg, wf[H:]) + bf_ref[...]
    zin = h_f + h_static
    wh1 = wh1_ref[...]
    hh = _relu(_dot(zin, wh1[:H]) + _dot(hglob_ref[...], wh1[H:])
               + bh1_ref[...])
    pred_ref[...] = _dot(hh, wh2_ref[...]) + bh2_ref[...]


def kernel(dynamic, static, mask, W_c0, b_c0, W_c1, b_c1, W_c2, b_c2, W_r0,
           W_r1, grn_W1, grn_b1, grn_W2, grn_b2, grn_Wg, grn_bg, grn_Ws,
           grn_bs, ln_g, ln_b, db, W_fuse, b_fuse, W_h1, b_h1, W_h2, b_h2):
    B, T, ND = dynamic.shape
    H = db.shape[1]
    NP = db.shape[0]

    wc0 = jnp.transpose(W_c0, (2, 1, 0))
    wc1 = jnp.transpose(W_c1, (2, 1, 0))
    wc2 = jnp.transpose(W_c2, (2, 1, 0))
    wr0 = W_r0[:, :, 0].T
    wr1 = W_r1[:, :, 0].T
    row = lambda v: v.reshape(1, -1)

    bb = _BB_TCN
    grid_a = (B // bb,)
    const3 = lambda shp: pl.BlockSpec(shp, lambda i: (0, 0, 0))
    const2 = lambda shp: pl.BlockSpec(shp, lambda i: (0, 0))
    hcur, hglob = pl.pallas_call(
        _tcn_kernel,
        grid=grid_a,
        in_specs=[
            pl.BlockSpec((bb, T, ND), lambda i: (i, 0, 0)),
            pl.BlockSpec((bb, T), lambda i: (i, 0)),
            const3(wc0.shape), const3(wc1.shape), const3(wc2.shape),
            const2(wr0.shape), const2(wr1.shape),
            const2((1, 64)), const2((1, 128)), const2((1, 128)),
        ],
        out_specs=[
            pl.BlockSpec((bb, H), lambda i: (i, 0)),
            pl.BlockSpec((bb, H), lambda i: (i, 0)),
        ],
        out_shape=[
            jax.ShapeDtypeStruct((B, H), jnp.float32),
            jax.ShapeDtypeStruct((B, H), jnp.float32),
        ],
    )(dynamic, mask, wc0, wc1, wc2, wr0, wr1,
      row(b_c0), row(b_c1), row(b_c2))

    bh = _BB_HEAD
    grid_b = (B // bh,)
    pred = pl.pallas_call(
        _head_kernel,
        grid=grid_b,
        in_specs=[
            pl.BlockSpec((bh, H), lambda i: (i, 0)),
            pl.BlockSpec((bh, H), lambda i: (i, 0)),
            pl.BlockSpec((bh, static.shape[1]), lambda i: (i, 0)),
            const2(db.shape),
            const2(grn_W1.shape), const2((1, 64)),
            const2(grn_W2.shape), const2((1, 128)),
            const2(grn_Wg.shape), const2((1, 256)),
            const2(grn_Ws.shape), const2((1, 128)),
            const2((1, 128)), const2((1, 128)),
            const2(W_fuse.shape), const2((1, 128)),
            const2(W_h1.shape), const2((1, 128)),
            const2(W_h2.shape), const2((1, 1)),
        ],
        out_specs=pl.BlockSpec((bh, 1), lambda i: (i, 0)),
        out_shape=jax.ShapeDtypeStruct((B, 1), jnp.float32),
    )(hcur, hglob, static, db, grn_W1, row(grn_b1), grn_W2, row(grn_b2),
      grn_Wg, row(grn_bg), grn_Ws, row(grn_bs), row(ln_g), row(ln_b),
      W_fuse, row(b_fuse), W_h1, row(b_h1), W_h2, row(b_h2))

    return pred[:, 0], hcur


# speedup vs baseline: 2.0827x; 1.3068x over previous
import jax
import jax.numpy as jnp
from jax.experimental import pallas as pl

_BB_TCN = 32
_BB_HEAD = 128
_TOPK = 5


def _relu(x):
    return jnp.maximum(x, 0.0)


def _dot(a, b):
    return jnp.dot(a, b, preferred_element_type=jnp.float32)


def _tcn_kernel(dyn_ref, mask_ref, mflat_ref, wc0_ref, wc1_ref, wc2_ref,
                wr0_ref, wr1_ref, b0_ref, b1_ref, b2_ref,
                hcur_ref, hglob_ref):
    R, T = mask_ref.shape
    RT = dyn_ref.shape[0]
    tmod = jax.lax.broadcasted_iota(jnp.int32, (RT, 1), 0) % T

    def tapcat(xf, d):
        c = xf.shape[1]

        def shifted(s):
            z = jnp.zeros((s, c), xf.dtype)
            xs = jnp.concatenate([z, xf[:-s]], axis=0)
            return jnp.where(tmod >= s, xs, 0.0)

        return jnp.concatenate([shifted(2 * d), shifted(d), xf], axis=1)

    x = dyn_ref[...]
    h = _relu(_dot(tapcat(x, 1), wc0_ref[...]) + b0_ref[...])
    x1 = _relu(h + _dot(x, wr0_ref[...]))
    h = _relu(_dot(tapcat(x1, 2), wc1_ref[...]) + b1_ref[...])
    x2 = _relu(h + _dot(x1, wr1_ref[...]))
    h = _relu(_dot(tapcat(x2, 4), wc2_ref[...]) + b2_ref[...])
    x3 = _relu(h + x2)

    m = mask_ref[...]
    msum = jnp.sum(m, axis=1, keepdims=True)
    lengths = jnp.maximum(msum.astype(jnp.int32) - 1, 0)
    colb = jax.lax.broadcasted_iota(jnp.int32, (R, RT), 1) // T
    rowi = jax.lax.broadcasted_iota(jnp.int32, (R, RT), 0)
    tcol = jax.lax.broadcasted_iota(jnp.int32, (R, RT), 1) % T
    bdiag = colb == rowi
    s_mask = jnp.where(bdiag, mflat_ref[...], 0.0)
    s_cur = (bdiag & (tcol == lengths)).astype(jnp.float32)
    sel = jnp.concatenate([s_mask, s_cur], axis=0)
    out2 = _dot(sel, x3)
    hglob_ref[...] = out2[:R] / jnp.maximum(msum, 1.0)
    hcur_ref[...] = out2[R:]


def _head_kernel(hcur_ref, hglob_ref, static_ref, db_ref, w1_ref, bb1_ref,
                 w2_ref, bb2_ref, wg_ref, bg_ref, ws_ref, bs_ref, lng_ref,
                 lnb_ref, wf_ref, bf_ref, wh1_ref, bh1_ref, wh2_ref, bh2_ref,
                 pred_ref):
    BB = hcur_ref.shape[0]
    H = hcur_ref.shape[1]
    db = db_ref[...]
    hc = hcur_ref[...]

    sim = jax.lax.dot_general(hc, db, (((1,), (1,)), ((), ())),
                              preferred_element_type=jnp.float32)
    s = sim
    v0 = None
    wmat = jnp.zeros(sim.shape, jnp.float32)
    z = None
    for k in range(_TOPK):
        mx = jnp.max(s, axis=1, keepdims=True)
        if k == 0:
            v0 = mx
            we = jnp.ones_like(mx)
        else:
            we = jnp.exp(mx - v0)
        hot = s == mx
        wmat = jnp.where(hot, we, wmat)
        s = jnp.where(hot, -1e30, s)
        z = we if k == 0 else z + we
    agg = _dot(wmat, db) / z

    st = static_ref[...]
    h1p = _dot(st, w1_ref[...]) + bb1_ref[...]
    h1 = jnp.where(h1p > 0, h1p, jnp.exp(jnp.minimum(h1p, 0.0)) - 1.0)
    h2 = _dot(h1, w2_ref[...]) + bb2_ref[...]
    g = _dot(h2, wg_ref[...]) + bg_ref[...]
    a = g[:, :H]
    bv = g[:, H:]
    gated = bv / (1.0 + jnp.exp(-a))
    skip = _dot(st, ws_ref[...]) + bs_ref[...]
    y = skip + gated
    mu = jnp.mean(y, axis=-1, keepdims=True)
    var = jnp.mean((y - mu) ** 2, axis=-1, keepdims=True)
    h_static = (y - mu) / jnp.sqrt(var + 1e-5) * lng_ref[...] + lnb_ref[...]

    wf = wf_ref[...]
    h_f = _dot(hc, wf[:H]) + _dot(agg, wf[H:]) + bf_ref[...]
    zin = h_f + h_static
    wh1 = wh1_ref[...]
    hh = _relu(_dot(zin, wh1[:H]) + _dot(hglob_ref[...], wh1[H:])
               + bh1_ref[...])
    pred_ref[...] = _dot(hh, wh2_ref[...]) + bh2_ref[...]


def kernel(dynamic, static, mask, W_c0, b_c0, W_c1, b_c1, W_c2, b_c2, W_r0,
           W_r1, grn_W1, grn_b1, grn_W2, grn_b2, grn_Wg, grn_bg, grn_Ws,
           grn_bs, ln_g, ln_b, db, W_fuse, b_fuse, W_h1, b_h1, W_h2, b_h2):
    B, T, ND = dynamic.shape
    H = db.shape[1]

    dyn_flat = dynamic.reshape(B * T, ND)
    mask_flat = mask.reshape(1, B * T)
    stk = lambda W: jnp.transpose(W, (2, 1, 0)).reshape(-1, W.shape[0])
    wc0 = stk(W_c0)
    wc1 = stk(W_c1)
    wc2 = stk(W_c2)
    wr0 = W_r0[:, :, 0].T
    wr1 = W_r1[:, :, 0].T
    row = lambda v: v.reshape(1, -1)

    bb = _BB_TCN
    grid_a = (B // bb,)
    const2 = lambda shp: pl.BlockSpec(shp, lambda i: (0, 0))
    hcur, hglob = pl.pallas_call(
        _tcn_kernel,
        grid=grid_a,
        in_specs=[
            pl.BlockSpec((bb * T, ND), lambda i: (i, 0)),
            pl.BlockSpec((bb, T), lambda i: (i, 0)),
            pl.BlockSpec((1, bb * T), lambda i: (0, i)),
            const2(wc0.shape), const2(wc1.shape), const2(wc2.shape),
            const2(wr0.shape), const2(wr1.shape),
            const2((1, 64)), const2((1, 128)), const2((1, 128)),
        ],
        out_specs=[
            pl.BlockSpec((bb, H), lambda i: (i, 0)),
            pl.BlockSpec((bb, H), lambda i: (i, 0)),
        ],
        out_shape=[
            jax.ShapeDtypeStruct((B, H), jnp.float32),
            jax.ShapeDtypeStruct((B, H), jnp.float32),
        ],
    )(dyn_flat, mask, mask_flat, wc0, wc1, wc2, wr0, wr1,
      row(b_c0), row(b_c1), row(b_c2))

    bh = _BB_HEAD
    grid_b = (B // bh,)
    pred = pl.pallas_call(
        _head_kernel,
        grid=grid_b,
        in_specs=[
            pl.BlockSpec((bh, H), lambda i: (i, 0)),
            pl.BlockSpec((bh, H), lambda i: (i, 0)),
            pl.BlockSpec((bh, static.shape[1]), lambda i: (i, 0)),
            const2(db.shape),
            const2(grn_W1.shape), const2((1, 64)),
            const2(grn_W2.shape), const2((1, 128)),
            const2(grn_Wg.shape), const2((1, 256)),
            const2(grn_Ws.shape), const2((1, 128)),
            const2((1, 128)), const2((1, 128)),
            const2(W_fuse.shape), const2((1, 128)),
            const2(W_h1.shape), const2((1, 128)),
            const2(W_h2.shape), const2((1, 1)),
        ],
        out_specs=pl.BlockSpec((bh, 1), lambda i: (i, 0)),
        out_shape=jax.ShapeDtypeStruct((B, 1), jnp.float32),
    )(hcur, hglob, static, db, grn_W1, row(grn_b1), grn_W2, row(grn_b2),
      grn_Wg, row(grn_bg), grn_Ws, row(grn_bs), row(ln_g), row(ln_b),
      W_fuse, row(b_fuse), W_h1, row(b_h1), W_h2, row(b_h2))

    return pred[:, 0], hcur
